# baseline (speedup 1.0000x reference)
import jax
import jax.numpy as jnp
from jax import lax
from jax.experimental import pallas as pl
from jax.experimental.pallas import tpu as pltpu

NUM_EMBEDDINGS = 8192
EMBED_DIM = 256

_BM = 512


def _quantize_body(idx_ref, table_ref, x_ref, out_ref):
    def body(i, carry):
        j = idx_ref[0, 0, i]
        q = table_ref[pl.ds(j, 1), :]
        x = x_ref[pl.ds(i, 1), :]
        out_ref[pl.ds(i, 1), :] = x + (q - x)
        return carry

    lax.fori_loop(0, _BM, body, 0, unroll=False)


def _quantize(table, idx, flat):
    n = idx.shape[0]
    grid = n // _BM
    return pl.pallas_call(
        _quantize_body,
        grid=(grid,),
        in_specs=[
            pl.BlockSpec((1, 1, _BM), lambda i: (i, 0, 0),
                         memory_space=pltpu.SMEM),
            pl.BlockSpec((NUM_EMBEDDINGS, EMBED_DIM), lambda i: (0, 0)),
            pl.BlockSpec((_BM, EMBED_DIM), lambda i: (i, 0)),
        ],
        out_specs=pl.BlockSpec((_BM, EMBED_DIM), lambda i: (i, 0)),
        out_shape=jax.ShapeDtypeStruct((n, EMBED_DIM), jnp.float32),
        compiler_params=pltpu.CompilerParams(
            vmem_limit_bytes=16 * 1024 * 1024,
        ),
    )(idx.reshape(grid, 1, _BM), table, flat)


def kernel(x, embeddings):
    d = embeddings.shape[0]
    flat = jnp.reshape(x, (-1, d))
    distances = (jnp.sum(flat ** 2, axis=1, keepdims=True)
                 - 2.0 * jnp.matmul(flat, embeddings)
                 + jnp.sum(embeddings ** 2, axis=0, keepdims=True))
    idx = jnp.argmax(-distances, axis=1).astype(jnp.int32)
    emb_seq, idx = lax.optimization_barrier((embeddings, idx))
    table = emb_seq.T.astype(jnp.bfloat16).astype(jnp.float32)
    out = _quantize(table, idx, flat)
    return jnp.reshape(out, x.shape)


# speedup vs baseline: 1.2042x; 1.2042x over previous
import jax
import jax.numpy as jnp
from jax import lax
from jax.experimental import pallas as pl
from jax.experimental.pallas import tpu as pltpu

NUM_EMBEDDINGS = 8192
EMBED_DIM = 256

_BM = 512


def _quantize_body(idx_ref, table_ref, x_ref, out_ref):
    def body(i, carry):
        j = idx_ref[0, 0, i]
        q = table_ref[pl.ds(j, 1), :]
        x = x_ref[pl.ds(i, 1), :]
        out_ref[pl.ds(i, 1), :] = x + (q - x)
        return carry

    lax.fori_loop(0, _BM, body, 0, unroll=8)


def _quantize(table, idx, flat):
    n = idx.shape[0]
    grid = n // _BM
    return pl.pallas_call(
        _quantize_body,
        grid=(grid,),
        in_specs=[
            pl.BlockSpec((1, 1, _BM), lambda i: (i, 0, 0),
                         memory_space=pltpu.SMEM),
            pl.BlockSpec((NUM_EMBEDDINGS, EMBED_DIM), lambda i: (0, 0)),
            pl.BlockSpec((_BM, EMBED_DIM), lambda i: (i, 0)),
        ],
        out_specs=pl.BlockSpec((_BM, EMBED_DIM), lambda i: (i, 0)),
        out_shape=jax.ShapeDtypeStruct((n, EMBED_DIM), jnp.float32),
        compiler_params=pltpu.CompilerParams(
            vmem_limit_bytes=16 * 1024 * 1024,
        ),
    )(idx.reshape(grid, 1, _BM), table, flat)


def kernel(x, embeddings):
    d = embeddings.shape[0]
    flat = jnp.reshape(x, (-1, d))
    distances = (jnp.sum(flat ** 2, axis=1, keepdims=True)
                 - 2.0 * jnp.matmul(flat, embeddings)
                 + jnp.sum(embeddings ** 2, axis=0, keepdims=True))
    idx = jnp.argmax(-distances, axis=1).astype(jnp.int32)
    emb_seq, idx = lax.optimization_barrier((embeddings, idx))
    table = emb_seq.T.astype(jnp.bfloat16).astype(jnp.float32)
    out = _quantize(table, idx, flat)
    return jnp.reshape(out, x.shape)
